# baseline (speedup 1.0000x reference)
import functools

import jax
import jax.numpy as jnp
from jax import lax
from jax.experimental import pallas as pl
from jax.experimental.pallas import tpu as pltpu
from jax.experimental.pallas import tpu_sc as plsc

_DIM = 32
_NC, _NS = 2, 16
_NW = _NC * _NS

_mesh = plsc.VectorSubcoreMesh(
    core_axis_name="c", subcore_axis_name="s",
    num_cores=_NC, num_subcores=_NS)


@functools.partial(jax.jit, static_argnums=(2,))
def _gather_rows(idx_flat, table, n_total):
  b_per_w = n_total // _NW
  chunk = 1280
  n_chunks = b_per_w // chunk

  @functools.partial(
      pl.kernel,
      out_type=jax.ShapeDtypeStruct((n_total, _DIM), jnp.float32),
      mesh=_mesh,
      scratch_types=[
          pltpu.VMEM((2, chunk), jnp.int32),
          pltpu.VMEM((2, chunk, _DIM), jnp.float32),
          pltpu.SemaphoreType.DMA((2,)),
          pltpu.SemaphoreType.DMA((2,)),
      ],
      compiler_params=pltpu.CompilerParams(use_tc_tiling_on_sc=False),
  )
  def gather_kernel(idx_hbm, table_hbm, out_hbm, idx_v, rows_v, gsem, ssem):
    wid = lax.axis_index("s") * _NC + lax.axis_index("c")
    base = wid * b_per_w

    def issue_gather(g, b):
      off = base + g * chunk
      pltpu.sync_copy(idx_hbm.at[pl.ds(off, chunk)], idx_v.at[b])
      return pltpu.async_copy(table_hbm.at[idx_v.at[b]], rows_v.at[b],
                              gsem.at[b])

    def issue_store(g, b):
      off = base + g * chunk
      return pltpu.async_copy(rows_v.at[b], out_hbm.at[pl.ds(off, chunk)],
                              ssem.at[b])

    gathers = {0: issue_gather(0, 0)}
    stores = {}
    for g in range(n_chunks):
      b = g % 2
      if g + 1 < n_chunks:
        if g >= 1:
          stores.pop(g - 1).wait()
        gathers[g + 1] = issue_gather(g + 1, 1 - b)
      gathers.pop(g).wait()
      stores[g] = issue_store(g, b)
    for g in sorted(stores):
      stores.pop(g).wait()

  return gather_kernel(idx_flat, table)


def kernel(i, table):
  flat = i.reshape(-1)
  out = _gather_rows(flat, table, flat.shape[0])
  return out.reshape(i.shape + (table.shape[-1],))


# speedup vs baseline: 1.0473x; 1.0473x over previous
import functools

import jax
import jax.numpy as jnp
from jax import lax
from jax.experimental import pallas as pl
from jax.experimental.pallas import tpu as pltpu
from jax.experimental.pallas import tpu_sc as plsc

_DIM = 32
_NC, _NS = 2, 16
_NW = _NC * _NS
_SB = 8
_BB = 128

_mesh = plsc.VectorSubcoreMesh(
    core_axis_name="c", subcore_axis_name="s",
    num_cores=_NC, num_subcores=_NS)


@functools.partial(jax.jit, static_argnums=(2, 3))
def _gather_rows(idx_t, table, s_total, b_total):
  n_units = s_total // _SB

  @functools.partial(
      pl.kernel,
      out_type=jax.ShapeDtypeStruct((s_total, b_total, _DIM), jnp.float32),
      mesh=_mesh,
      scratch_types=[
          pltpu.VMEM((2, _SB, _BB), jnp.int32),
          pltpu.VMEM((2, _SB, _BB, _DIM), jnp.float32),
          pltpu.SemaphoreType.DMA((2,)),
          pltpu.SemaphoreType.DMA((2,)),
      ],
      compiler_params=pltpu.CompilerParams(use_tc_tiling_on_sc=False),
  )
  def gather_kernel(idx_hbm, table_hbm, out_hbm, idx_v, rows_v, gsem, ssem):
    wid = lax.axis_index("s") * _NC + lax.axis_index("c")
    b0 = wid * _BB

    def issue_gather(k, b):
      pltpu.sync_copy(idx_hbm.at[pl.ds(k * _SB, _SB), pl.ds(b0, _BB)],
                      idx_v.at[b])
      return [
          pltpu.async_copy(table_hbm.at[idx_v.at[b, si]], rows_v.at[b, si],
                           gsem.at[b])
          for si in range(_SB)
      ]

    def issue_store(k, b):
      return pltpu.async_copy(
          rows_v.at[b],
          out_hbm.at[pl.ds(k * _SB, _SB), pl.ds(b0, _BB), :],
          ssem.at[b])

    gathers = {0: issue_gather(0, 0)}
    stores = {}
    for k in range(n_units):
      b = k % 2
      if k + 1 < n_units:
        if k >= 1:
          stores.pop(k - 1).wait()
        gathers[k + 1] = issue_gather(k + 1, 1 - b)
      for h in gathers.pop(k):
        h.wait()
      stores[k] = issue_store(k, b)
    for k in sorted(stores):
      stores.pop(k).wait()

  return gather_kernel(idx_t, table)


def kernel(i, table):
  b_total, s_total = i.shape
  out_t = _gather_rows(i.T, table, s_total, b_total)
  return out_t.transpose(1, 0, 2)
